# baseline (speedup 1.0000x reference)
import functools
import jax
import jax.numpy as jnp
from jax import lax
from jax.experimental import pallas as pl
from jax.experimental.pallas import tpu as pltpu
from jax.experimental.pallas import tpu_sc as plsc

NT = 64
NB = 16
TOT = 32768
CT = 256
LN2 = 0.6931471805599453

_GDN = lax.GatherDimensionNumbers(
    offset_dims=(), collapsed_slice_dims=(0,), start_index_map=(0,))


def _perm(v, idx):
    return lax.gather(v, idx[:, None], _GDN, (1,),
                      mode=lax.GatherScatterMode.PROMISE_IN_BOUNDS)


def _xors():
    lanes = lax.iota(jnp.int32, 16)
    return [lanes ^ k for k in (8, 4, 2, 1)]


def _lane_max(v, xors):
    for x in xors:
        v = jnp.maximum(v, _perm(v, x))
    return v


def _lane_sum(v, xors):
    for x in xors:
        v = v + _perm(v, x)
    return v


def kernel(emissions, transitions, head_transitions, tail_transitions,
           cu_seqlens):
    em = emissions.reshape(TOT * NT)
    tr = transitions.reshape(NT * NT)
    hd = head_transitions.reshape(NT)
    tl = tail_transitions.reshape(NT)
    cu = jnp.pad(cu_seqlens.astype(jnp.int32), (0, 31))

    mesh = plsc.VectorSubcoreMesh(core_axis_name="c", subcore_axis_name="s")

    @functools.partial(
        pl.kernel, mesh=mesh,
        out_type=jax.ShapeDtypeStruct((NB, 16), jnp.float32),
        scratch_types=[
            pltpu.VMEM((NT * NT,), jnp.float32),
            pltpu.VMEM((NT,), jnp.float32),
            pltpu.VMEM((NT,), jnp.float32),
            pltpu.VMEM((48,), jnp.int32),
            pltpu.VMEM((CT * NT,), jnp.float32),
            pltpu.VMEM((16,), jnp.float32),
        ],
    )
    def sck(em_h, tr_h, hd_h, tl_h, cu_h, out_h, Ev, ehv, etv, cuv, ebuf,
            orow):
        cid = lax.axis_index("c")
        sid = lax.axis_index("s")
        wid = sid * 2 + cid

        @pl.when(wid < NB)
        def _():
            pltpu.sync_copy(cu_h, cuv)
            pltpu.sync_copy(tr_h, Ev)
            pltpu.sync_copy(hd_h, ehv)
            pltpu.sync_copy(tl_h, etv)

            def eexp(i, c):
                Ev[pl.ds(i * 16, 16)] = jnp.exp(Ev[pl.ds(i * 16, 16)])
                return c
            lax.fori_loop(0, NT * NT // 16, eexp, 0)

            start = cuv[pl.ds(wid, 16)][0]
            ln = cuv[pl.ds(wid + 1, 16)][0] - start
            nch = lax.div(ln + (CT - 1), CT)

            xors = _xors()
            ehr = [jnp.exp(ehv[pl.ds(v * 16, 16)]) for v in range(4)]
            IDX = [jnp.full((16,), l, jnp.int32) for l in range(16)]
            k127 = jnp.full((16,), 127, jnp.int32)
            k254 = jnp.full((16,), 254, jnp.int32)

            def chunk(ci, carry):
                off = jnp.minimum(start + ci * CT, TOT - CT)
                pltpu.sync_copy(em_h.at[pl.ds(off * NT, CT * NT)], ebuf)
                hi = jnp.minimum(ln - ci * CT, CT)

                def step(t, c2):
                    ps = list(c2[:4])
                    cf = c2[4]
                    tt = ci * CT + t
                    ee = [jnp.exp(ebuf[pl.ds(t * NT + v * 16, 16)])
                          for v in range(4)]
                    accs = [jnp.zeros((16,), jnp.float32) for _ in range(4)]
                    for i in range(NT):
                        bc = _perm(ps[i // 16], IDX[i % 16])
                        for v in range(4):
                            accs[v] = accs[v] + bc * Ev[
                                pl.ds(i * NT + v * 16, 16)]
                    is0 = tt == 0
                    q = [jnp.where(is0, ehr[v], accs[v]) * ee[v]
                         for v in range(4)]
                    mx = _lane_max(jnp.maximum(jnp.maximum(q[0], q[1]),
                                               jnp.maximum(q[2], q[3])), xors)
                    bits = lax.bitcast_convert_type(mx, jnp.int32)
                    ef = lax.shift_right_logical(bits, 23) & 0xFF
                    scale = lax.bitcast_convert_type(
                        lax.shift_left(k254 - ef, 23), jnp.float32)
                    pn = [q[v] * scale for v in range(4)]
                    cf = cf + (ef - k127).astype(jnp.float32)
                    return (pn[0], pn[1], pn[2], pn[3], cf)

                return lax.fori_loop(0, hi, step, carry)

            z = jnp.zeros((16,), jnp.float32)
            fin = lax.fori_loop(0, nch, chunk, (z, z, z, z, z))
            cf = fin[4]
            etr = [jnp.exp(etv[pl.ds(v * 16, 16)]) for v in range(4)]
            s4 = (fin[0] * etr[0] + fin[1] * etr[1] + fin[2] * etr[2]
                  + fin[3] * etr[3])
            s = _lane_sum(s4, xors)
            sb = lax.bitcast_convert_type(s, jnp.int32)
            se = (lax.shift_right_logical(sb, 23) & 0xFF) - k127
            m = lax.bitcast_convert_type(
                (sb & 0x7FFFFF) | 0x3F800000, jnp.float32)
            zq = (m - 1.0) / (m + 1.0)
            z2 = zq * zq
            lnm = 2.0 * zq * (1.0 + z2 * (1.0 / 3 + z2 * (
                1.0 / 5 + z2 * (1.0 / 7 + z2 * (1.0 / 9 + z2 / 11)))))
            logz = (cf + se.astype(jnp.float32)) * LN2 + lnm
            orow[...] = logz
            pltpu.sync_copy(orow, out_h.at[wid])

    out = sck(em, tr, hd, tl, cu)
    return out[:, 0:1]


# speedup vs baseline: 2.9828x; 2.9828x over previous
import jax
import jax.numpy as jnp
from jax import lax
from jax.experimental import pallas as pl
from jax.experimental.pallas import tpu as pltpu

NT = 64
NB = 16
TOT = 32768
CHUNK = 512
GROUP = 8
LN2 = 0.6931471805599453


def _crf_body(cu_ref, lens_ref, em_hbm, trans_ref, head_ref, tail_ref,
              out_ref, embuf, sem):
    E = jnp.exp(trans_ref[...]).astype(jnp.bfloat16)
    eh = jnp.exp(head_ref[...])
    et = jnp.exp(tail_ref[...])
    lens = lens_ref[...]

    def mx(b, m):
        return jnp.maximum(m, cu_ref[b + 1] - cu_ref[b])
    maxlen = lax.fori_loop(0, NB, mx, jnp.int32(0))
    nch = lax.div(maxlen + (CHUNK - 1), CHUNK)

    def issue(ci, buf):
        for b in range(NB):
            off = jnp.minimum(cu_ref[b] + ci * CHUNK, TOT - CHUNK)
            pltpu.make_async_copy(
                em_hbm.at[pl.ds(off, CHUNK), :],
                embuf.at[buf, :, b, :],
                sem.at[buf],
            ).start()

    def wait(buf):
        for b in range(NB):
            pltpu.make_async_copy(
                em_hbm.at[pl.ds(0, CHUNK), :],
                embuf.at[buf, :, b, :],
                sem.at[buf],
            ).wait()

    issue(0, 0)

    def chunk_body(ci, carry):
        buf = lax.rem(ci, 2)

        @pl.when(ci + 1 < nch)
        def _():
            issue(ci + 1, 1 - buf)

        wait(buf)

        def group(g, c2):
            p, cf, psnap, cfsnap = c2
            q = p
            for k in range(GROUP):
                t = GROUP * g + k
                tg = ci * CHUNK + t
                eem = jnp.exp(embuf[buf, t])
                q0 = lax.dot_general(q.astype(jnp.bfloat16), E,
                                     (((1,), (0,)), ((), ())),
                                     preferred_element_type=jnp.float32)
                q = jnp.where(tg == 0, eh, q0) * eem
                hit = tg == (lens - 1)
                psnap = jnp.where(hit, q, psnap)
                cfsnap = jnp.where(hit, cf, cfsnap)
            m = jnp.max(q, axis=1, keepdims=True)
            bits = lax.bitcast_convert_type(m, jnp.int32)
            ef = lax.shift_right_logical(bits, 23) & 0xFF
            scale = lax.bitcast_convert_type(
                lax.shift_left(254 - ef, 23), jnp.float32)
            p = q * scale
            cf = cf + (ef - 127).astype(jnp.float32)
            return (p, cf, psnap, cfsnap)

        return lax.fori_loop(0, CHUNK // GROUP, group, carry, unroll=2)

    init = (jnp.zeros((NB, NT), jnp.float32), jnp.zeros((NB, 1), jnp.float32),
            jnp.zeros((NB, NT), jnp.float32), jnp.zeros((NB, 1), jnp.float32))
    p, cf, psnap, cfsnap = lax.fori_loop(0, nch, chunk_body, init)
    s = jnp.sum(psnap * et, axis=1, keepdims=True)
    out_ref[...] = jnp.log(s) + cfsnap * LN2


def kernel(emissions, transitions, head_transitions, tail_transitions,
           cu_seqlens):
    em = emissions.reshape(TOT, NT)
    trans = transitions.reshape(NT, NT)
    head = head_transitions.reshape(1, NT)
    tail = tail_transitions.reshape(1, NT)
    cu = cu_seqlens.astype(jnp.int32)
    lens = (cu[1:] - cu[:-1]).reshape(NB, 1)
    return pl.pallas_call(
        _crf_body,
        out_shape=jax.ShapeDtypeStruct((NB, 1), jnp.float32),
        in_specs=[
            pl.BlockSpec(memory_space=pltpu.SMEM),
            pl.BlockSpec(memory_space=pltpu.VMEM),
            pl.BlockSpec(memory_space=pltpu.MemorySpace.HBM),
            pl.BlockSpec(memory_space=pltpu.VMEM),
            pl.BlockSpec(memory_space=pltpu.VMEM),
            pl.BlockSpec(memory_space=pltpu.VMEM),
        ],
        out_specs=pl.BlockSpec(memory_space=pltpu.VMEM),
        scratch_shapes=[
            pltpu.VMEM((2, CHUNK, NB, NT), jnp.float32),
            pltpu.SemaphoreType.DMA((2,)),
        ],
    )(cu, lens, em, trans, head, tail)


# speedup vs baseline: 3.0050x; 1.0074x over previous
import jax
import jax.numpy as jnp
from jax import lax
from jax.experimental import pallas as pl
from jax.experimental.pallas import tpu as pltpu

NT = 64
NB = 16
TOT = 32768
CHUNK = 512
GROUP = 8
LN2 = 0.6931471805599453


def _crf_body(cu_ref, lens_ref, em_hbm, trans_ref, head_ref, tail_ref,
              out_ref, embuf, sem):
    E = jnp.exp(trans_ref[...]).astype(jnp.bfloat16)
    eh = jnp.exp(head_ref[...])
    et = jnp.exp(tail_ref[...])
    lens = lens_ref[...]

    def mx(b, m):
        return jnp.maximum(m, cu_ref[b + 1] - cu_ref[b])
    maxlen = lax.fori_loop(0, NB, mx, jnp.int32(0))
    nch = lax.div(maxlen + (CHUNK - 1), CHUNK)

    def issue(ci, buf):
        for b in range(NB):
            off = jnp.minimum(cu_ref[b] + ci * CHUNK, TOT - CHUNK)
            pltpu.make_async_copy(
                em_hbm.at[pl.ds(off, CHUNK), :],
                embuf.at[buf, :, b, :],
                sem.at[buf],
            ).start()

    def wait(buf):
        for b in range(NB):
            pltpu.make_async_copy(
                em_hbm.at[pl.ds(0, CHUNK), :],
                embuf.at[buf, :, b, :],
                sem.at[buf],
            ).wait()

    issue(0, 0)

    def chunk_body(ci, carry):
        buf = lax.rem(ci, 2)

        @pl.when(ci + 1 < nch)
        def _():
            issue(ci + 1, 1 - buf)

        wait(buf)

        def group(g, c2):
            qs = [c2[0], c2[1]]
            cfs = [c2[2], c2[3]]
            psn = [c2[4], c2[5]]
            cfsn = [c2[6], c2[7]]
            lh = [lens[0:8], lens[8:16]]
            for k in range(GROUP):
                t = GROUP * g + k
                tg = ci * CHUNK + t
                eem = jnp.exp(embuf[buf, t])
                eh_ = [eem[0:8] * eh, eem[8:16] * eh]
                for h in range(2):
                    q0 = lax.dot_general(qs[h].astype(jnp.bfloat16), E,
                                         (((1,), (0,)), ((), ())),
                                         preferred_element_type=jnp.float32)
                    eslice = eem[0:8] if h == 0 else eem[8:16]
                    qs[h] = jnp.where(tg == 0, eh_[h], q0 * eslice)
                    hit = tg == (lh[h] - 1)
                    psn[h] = jnp.where(hit, qs[h], psn[h])
                    cfsn[h] = jnp.where(hit, cfs[h], cfsn[h])
            for h in range(2):
                m = jnp.max(qs[h], axis=1, keepdims=True)
                bits = lax.bitcast_convert_type(m, jnp.int32)
                ef = lax.shift_right_logical(bits, 23) & 0xFF
                scale = lax.bitcast_convert_type(
                    lax.shift_left(254 - ef, 23), jnp.float32)
                qs[h] = qs[h] * scale
                cfs[h] = cfs[h] + (ef - 127).astype(jnp.float32)
            return (qs[0], qs[1], cfs[0], cfs[1],
                    psn[0], psn[1], cfsn[0], cfsn[1])

        return lax.fori_loop(0, CHUNK // GROUP, group, carry, unroll=2)

    zv = jnp.zeros((NB // 2, NT), jnp.float32)
    zc = jnp.zeros((NB // 2, 1), jnp.float32)
    fin = lax.fori_loop(0, nch, chunk_body,
                        (zv, zv, zc, zc, zv, zv, zc, zc))
    psnap = jnp.concatenate([fin[4], fin[5]], axis=0)
    cfsnap = jnp.concatenate([fin[6], fin[7]], axis=0)
    s = jnp.sum(psnap * et, axis=1, keepdims=True)
    out_ref[...] = jnp.log(s) + cfsnap * LN2


def kernel(emissions, transitions, head_transitions, tail_transitions,
           cu_seqlens):
    em = emissions.reshape(TOT, NT)
    trans = transitions.reshape(NT, NT)
    head = head_transitions.reshape(1, NT)
    tail = tail_transitions.reshape(1, NT)
    cu = cu_seqlens.astype(jnp.int32)
    lens = (cu[1:] - cu[:-1]).reshape(NB, 1)
    return pl.pallas_call(
        _crf_body,
        out_shape=jax.ShapeDtypeStruct((NB, 1), jnp.float32),
        in_specs=[
            pl.BlockSpec(memory_space=pltpu.SMEM),
            pl.BlockSpec(memory_space=pltpu.VMEM),
            pl.BlockSpec(memory_space=pltpu.MemorySpace.HBM),
            pl.BlockSpec(memory_space=pltpu.VMEM),
            pl.BlockSpec(memory_space=pltpu.VMEM),
            pl.BlockSpec(memory_space=pltpu.VMEM),
        ],
        out_specs=pl.BlockSpec(memory_space=pltpu.VMEM),
        scratch_shapes=[
            pltpu.VMEM((2, CHUNK, NB, NT), jnp.float32),
            pltpu.SemaphoreType.DMA((2,)),
        ],
    )(cu, lens, em, trans, head, tail)
